# baseline (speedup 1.0000x reference)
import functools

import jax
import jax.numpy as jnp
import numpy as np
from jax import lax
from jax.experimental import pallas as pl
from jax.experimental.pallas import tpu as pltpu
from jax.experimental.pallas import tpu_sc as plsc


def _mm_bias_body(a_ref, w_ref, b_ref, o_ref):
    o_ref[...] = (
        jnp.dot(a_ref[...], w_ref[...], preferred_element_type=jnp.float32)
        + b_ref[...]
    )


def _mm_halves_body(a_ref, w_ref, o0_ref, o1_ref):
    y = jnp.dot(a_ref[...], w_ref[...], preferred_element_type=jnp.float32)
    half = y.shape[1] // 2

    def pack_half(off):
        words = []
        for g in range(half // 32):
            blk_a = y[:, off + 32 * g: off + 32 * g + 16]
            blk_b = y[:, off + 32 * g + 16: off + 32 * g + 32]
            a_bits = lax.bitcast_convert_type(
                blk_a.astype(jnp.bfloat16), jnp.uint16).astype(jnp.uint32)
            b_bits = lax.bitcast_convert_type(
                blk_b.astype(jnp.bfloat16), jnp.uint16).astype(jnp.uint32)
            words.append(a_bits | (b_bits << 16))
        return jnp.concatenate(words, axis=1)

    o0_ref[...] = pack_half(0)
    o1_ref[...] = pack_half(half)


def _resid_mm_body(h_ref, p0_ref, p1_ref, w_ref, b_ref, o_ref):
    half = w_ref.shape[0] // 2
    aggr_mm = (
        jnp.dot(p0_ref[...], w_ref[:half, :], preferred_element_type=jnp.float32)
        + jnp.dot(p1_ref[...], w_ref[half:, :], preferred_element_type=jnp.float32)
    )
    o_ref[...] = h_ref[...] + aggr_mm + b_ref[...]


def _mlp_body(h_ref, w1_ref, b1_ref, w2_ref, b2_ref, o_ref):
    hid = jnp.maximum(
        jnp.dot(h_ref[...], w1_ref[...], preferred_element_type=jnp.float32)
        + b1_ref[...],
        0.0,
    )
    o_ref[...] = (
        jnp.dot(hid, w2_ref[...], preferred_element_type=jnp.float32) + b2_ref[...]
    )


def _mm_bias(a, w, b, bm):
    m, k = a.shape
    dout = w.shape[1]
    return pl.pallas_call(
        _mm_bias_body,
        grid=(m // bm,),
        in_specs=[
            pl.BlockSpec((bm, k), lambda i: (i, 0)),
            pl.BlockSpec((k, dout), lambda i: (0, 0)),
            pl.BlockSpec((1, dout), lambda i: (0, 0)),
        ],
        out_specs=pl.BlockSpec((bm, dout), lambda i: (i, 0)),
        out_shape=jax.ShapeDtypeStruct((m, dout), jnp.float32),
    )(a, w, b.reshape(1, dout))


def _mm_halves(a, w, bm):
    m, k = a.shape
    dout = w.shape[1]
    quarter = dout // 4
    return pl.pallas_call(
        _mm_halves_body,
        grid=(m // bm,),
        in_specs=[
            pl.BlockSpec((bm, k), lambda i: (i, 0)),
            pl.BlockSpec((k, dout), lambda i: (0, 0)),
        ],
        out_specs=(
            pl.BlockSpec((bm, quarter), lambda i: (i, 0)),
            pl.BlockSpec((bm, quarter), lambda i: (i, 0)),
        ),
        out_shape=(
            jax.ShapeDtypeStruct((m, quarter), jnp.uint32),
            jax.ShapeDtypeStruct((m, quarter), jnp.uint32),
        ),
    )(a, w)


def _resid_mm(h, p0, p1, w, b, bm):
    m, k = h.shape
    dout = w.shape[1]
    half = k // 2
    return pl.pallas_call(
        _resid_mm_body,
        grid=(m // bm,),
        in_specs=[
            pl.BlockSpec((bm, k), lambda i: (i, 0)),
            pl.BlockSpec((bm, half), lambda i: (i, 0)),
            pl.BlockSpec((bm, half), lambda i: (i, 0)),
            pl.BlockSpec((k, dout), lambda i: (0, 0)),
            pl.BlockSpec((1, dout), lambda i: (0, 0)),
        ],
        out_specs=pl.BlockSpec((bm, dout), lambda i: (i, 0)),
        out_shape=jax.ShapeDtypeStruct((m, dout), jnp.float32),
    )(h, p0, p1, w, b.reshape(1, dout))


def _mlp(h, w1, b1, w2, b2, bm):
    m, k = h.shape
    dh = w1.shape[1]
    dout = w2.shape[1]
    return pl.pallas_call(
        _mlp_body,
        grid=(m // bm,),
        in_specs=[
            pl.BlockSpec((bm, k), lambda i: (i, 0)),
            pl.BlockSpec((k, dh), lambda i: (0, 0)),
            pl.BlockSpec((1, dh), lambda i: (0, 0)),
            pl.BlockSpec((dh, dout), lambda i: (0, 0)),
            pl.BlockSpec((1, dout), lambda i: (0, 0)),
        ],
        out_specs=pl.BlockSpec((bm, dout), lambda i: (i, 0)),
        out_shape=jax.ShapeDtypeStruct((m, dout), jnp.float32),
    )(h, w1, b1.reshape(1, dh), w2, b2.reshape(1, dout))


_NC = 2
_NS = 16
_LANES = 16


def _make_message_pass(n, dh, e, chunk):
    assert e % _NS == 0
    e_per_t = e // _NS
    assert e_per_t % chunk == 0 and chunk % 8 == 0 and chunk <= 128
    n_chunks = e_per_t // chunk
    assert n_chunks % 2 == 0
    np_rows = ((n + 8 * _NS - 1) // (8 * _NS)) * (8 * _NS)
    rows_per_tile = np_rows // _NS
    nvec = dh // _LANES
    unroll = 8
    assert chunk % unroll == 0

    mesh = plsc.VectorSubcoreMesh(core_axis_name="c", subcore_axis_name="s")

    @functools.partial(
        pl.kernel,
        mesh=mesh,
        compiler_params=pltpu.CompilerParams(
            use_tc_tiling_on_sc=False, needs_layout_passes=False),
        out_type=jax.ShapeDtypeStruct((_NC, np_rows, dh), jnp.float32),
        scratch_types=(
            [
                pltpu.VMEM_SHARED((np_rows, dh), jnp.float32),
                pltpu.VMEM((n_chunks, chunk), jnp.int32),
            ]
            + [pltpu.VMEM((chunk,), jnp.int32)] * 4
            + [pltpu.VMEM((chunk, dh // 2), jnp.uint32)] * 4
            + [pltpu.VMEM((chunk, dh // 2), jnp.uint32)] * 4
            + [pltpu.VMEM((chunk, dh), jnp.float32)] * 4
            + [pltpu.SemaphoreType.DMA] * 16
        ),
    )
    def msg_kernel(ht0_hbm, ht1_hbm, et0_hbm, et1_hbm, src_hbm, dst_hbm,
                   z_hbm, out_hbm, aggr, src_v, *rest):
        dbuf = rest[0:4]
        erows = rest[4:8]
        hrows = rest[8:12]
        mbuf = rest[12:16]
        sem_d = rest[16:20]
        sem_e = rest[20:24]
        sem_h = rest[24:28]
        sem_s = rest[28:32]
        core = lax.axis_index("c")
        sub = lax.axis_index("s")

        pltpu.sync_copy(
            z_hbm.at[pl.ds(sub * rows_per_tile, rows_per_tile)],
            aggr.at[pl.ds(sub * rows_per_tile, rows_per_tile)],
        )
        plsc.subcore_barrier()

        pltpu.sync_copy(src_hbm.at[pl.ds(sub * n_chunks, n_chunks)], src_v)

        def run_edges(ht_hbm, et_hbm):
            def issue_loads(g, b):
                base = sub * e_per_t + g * chunk
                pltpu.async_copy(dst_hbm.at[pl.ds(base, chunk)], dbuf[b],
                                 sem_d[b])
                pltpu.async_copy(et_hbm.at[pl.ds(base, chunk)], erows[b],
                                 sem_e[b])
                pltpu.async_copy(ht_hbm.at[pl.ds(0, chunk)], hrows[b],
                                 sem_h[b])

            def wait_loads(g, b):
                base = sub * e_per_t + g * chunk
                pltpu.make_async_copy(dst_hbm.at[pl.ds(base, chunk)], dbuf[b],
                                      sem_d[b]).wait()
                pltpu.make_async_copy(et_hbm.at[pl.ds(base, chunk)], erows[b],
                                      sem_e[b]).wait()
                pltpu.make_async_copy(ht_hbm.at[pl.ds(0, chunk)], hrows[b],
                                      sem_h[b]).wait()

            def wait_scatter(b):
                pltpu.make_async_copy(mbuf[b], aggr.at[dbuf[b]],
                                      sem_s[b]).wait()

            def step(g, b):
                pb = (b + 2) % 4
                wait_scatter(pb)

                @pl.when(g + 2 < n_chunks)
                def _():
                    issue_loads(g + 2, pb)

                wait_loads(g, b)

                eb = erows[b]
                hb = hrows[b]
                mb = mbuf[b]

                def edge_body(i, _):
                    for u in range(unroll):
                        ii = i * unroll + u
                        for j in range(dh // 32):
                            sw = pl.ds(j * _LANES, _LANES)
                            wh = hb[ii, sw]
                            we = eb[ii, sw]
                            h_lo = plsc.bitcast(wh << 16, jnp.float32)
                            h_hi = plsc.bitcast(
                                wh & jnp.uint32(0xFFFF0000), jnp.float32)
                            e_lo = plsc.bitcast(we << 16, jnp.float32)
                            e_hi = plsc.bitcast(
                                we & jnp.uint32(0xFFFF0000), jnp.float32)
                            mb[ii, pl.ds(j * 32, _LANES)] = jnp.maximum(
                                h_lo + e_lo, 0.0)
                            mb[ii, pl.ds(j * 32 + _LANES, _LANES)] = (
                                jnp.maximum(h_hi + e_hi, 0.0))
                    return 0
                lax.fori_loop(0, chunk // unroll, edge_body, 0)

                pltpu.async_copy(
                    mb, aggr.at[pl.ds(sub * rows_per_tile, chunk)], sem_s[b])

            pltpu.async_copy(z_hbm.at[pl.ds(0, chunk)], mbuf[2], sem_s[2])
            pltpu.async_copy(z_hbm.at[pl.ds(0, chunk)], mbuf[3], sem_s[3])
            issue_loads(0, 0)
            issue_loads(1, 1)

            def pipe_body(gp, _):
                for b in range(4):
                    step(gp * 4 + b, b)
                return 0

            lax.fori_loop(0, n_chunks // 4, pipe_body, 0)
            for t in range(n_chunks % 4):
                step(n_chunks - (n_chunks % 4) + t, t)
            for t in range(1, 3):
                wait_scatter((n_chunks - t) % 4)

        @pl.when(core == 0)
        def _():
            run_edges(ht0_hbm, et0_hbm)

        @pl.when(core == 1)
        def _():
            run_edges(ht1_hbm, et1_hbm)

        plsc.subcore_barrier()
        pltpu.sync_copy(
            aggr.at[pl.ds(sub * rows_per_tile, rows_per_tile)],
            out_hbm.at[core, pl.ds(sub * rows_per_tile, rows_per_tile)],
        )

    return msg_kernel, np_rows


def kernel(x, edge_index, edge_attr, W_in, b_in, W_nbr, W_e, W_self, b_conv,
           W_m1, b_m1, W_m2, b_m2):
    n, d = x.shape
    e = edge_index.shape[1]
    num_layers = W_nbr.shape[0]
    src = edge_index[0]
    dst = edge_index[1]

    bm = 1000
    be = 4000
    chunk = 80

    msg_pass, np_rows = _make_message_pass(n, d // 2, e, chunk=chunk)
    src = src.reshape(e // chunk, chunk)

    zeros_np = jnp.zeros((np_rows, d // 2), jnp.float32)

    ets = [_mm_halves(edge_attr, W_e[l], be) for l in range(num_layers)]

    h = _mm_bias(x, W_in, b_in, bm)
    for l in range(num_layers):
        ht0, ht1 = _mm_halves(h, W_nbr[l], bm)
        et0, et1 = ets[l]
        partials = msg_pass(ht0, ht1, et0, et1, src, dst, zeros_np)
        p0 = partials[0, :n]
        p1 = partials[1, :n]
        h = _resid_mm(h, p0, p1, W_self[l], b_conv[l], bm)

    out = _mlp(h, W_m1, b_m1, W_m2, b_m2, bm)
    return out[:, 0]


# speedup vs baseline: 1.1457x; 1.1457x over previous
import functools

import jax
import jax.numpy as jnp
import numpy as np
from jax import lax
from jax.experimental import pallas as pl
from jax.experimental.pallas import tpu as pltpu
from jax.experimental.pallas import tpu_sc as plsc


def _mm_bias_body(a_ref, w_ref, b_ref, o_ref):
    o_ref[...] = (
        jnp.dot(a_ref[...], w_ref[...], preferred_element_type=jnp.float32)
        + b_ref[...]
    )


def _mm_halves_body(a_ref, w_ref, o0_ref, o1_ref):
    y = jnp.dot(a_ref[...], w_ref[...], preferred_element_type=jnp.float32)
    half = y.shape[1] // 2

    def pack_half(off):
        words = []
        for g in range(half // 32):
            blk_a = y[:, off + 32 * g: off + 32 * g + 16]
            blk_b = y[:, off + 32 * g + 16: off + 32 * g + 32]
            a_bits = lax.bitcast_convert_type(
                blk_a.astype(jnp.bfloat16), jnp.uint16).astype(jnp.uint32)
            b_bits = lax.bitcast_convert_type(
                blk_b.astype(jnp.bfloat16), jnp.uint16).astype(jnp.uint32)
            words.append(a_bits | (b_bits << 16))
        return jnp.concatenate(words, axis=1)

    o0_ref[...] = pack_half(0)
    o1_ref[...] = pack_half(half)


def _resid_mm_body(h_ref, p0_ref, p1_ref, w_ref, b_ref, o_ref):
    half = w_ref.shape[0] // 2
    aggr_mm = (
        jnp.dot(p0_ref[...], w_ref[:half, :], preferred_element_type=jnp.float32)
        + jnp.dot(p1_ref[...], w_ref[half:, :], preferred_element_type=jnp.float32)
    )
    o_ref[...] = h_ref[...] + aggr_mm + b_ref[...]


def _mlp_body(h_ref, w1_ref, b1_ref, w2_ref, b2_ref, o_ref):
    hid = jnp.maximum(
        jnp.dot(h_ref[...], w1_ref[...], preferred_element_type=jnp.float32)
        + b1_ref[...],
        0.0,
    )
    o_ref[...] = (
        jnp.dot(hid, w2_ref[...], preferred_element_type=jnp.float32) + b2_ref[...]
    )


def _mm_bias(a, w, b, bm):
    m, k = a.shape
    dout = w.shape[1]
    return pl.pallas_call(
        _mm_bias_body,
        grid=(m // bm,),
        in_specs=[
            pl.BlockSpec((bm, k), lambda i: (i, 0)),
            pl.BlockSpec((k, dout), lambda i: (0, 0)),
            pl.BlockSpec((1, dout), lambda i: (0, 0)),
        ],
        out_specs=pl.BlockSpec((bm, dout), lambda i: (i, 0)),
        out_shape=jax.ShapeDtypeStruct((m, dout), jnp.float32),
    )(a, w, b.reshape(1, dout))


def _mm_halves(a, w, bm):
    m, k = a.shape
    dout = w.shape[1]
    quarter = dout // 4
    return pl.pallas_call(
        _mm_halves_body,
        grid=(m // bm,),
        in_specs=[
            pl.BlockSpec((bm, k), lambda i: (i, 0)),
            pl.BlockSpec((k, dout), lambda i: (0, 0)),
        ],
        out_specs=(
            pl.BlockSpec((bm, quarter), lambda i: (i, 0)),
            pl.BlockSpec((bm, quarter), lambda i: (i, 0)),
        ),
        out_shape=(
            jax.ShapeDtypeStruct((m, quarter), jnp.uint32),
            jax.ShapeDtypeStruct((m, quarter), jnp.uint32),
        ),
    )(a, w)


def _resid_mm(h, p0, p1, w, b, bm):
    m, k = h.shape
    dout = w.shape[1]
    half = k // 2
    return pl.pallas_call(
        _resid_mm_body,
        grid=(m // bm,),
        in_specs=[
            pl.BlockSpec((bm, k), lambda i: (i, 0)),
            pl.BlockSpec((bm, half), lambda i: (i, 0)),
            pl.BlockSpec((bm, half), lambda i: (i, 0)),
            pl.BlockSpec((k, dout), lambda i: (0, 0)),
            pl.BlockSpec((1, dout), lambda i: (0, 0)),
        ],
        out_specs=pl.BlockSpec((bm, dout), lambda i: (i, 0)),
        out_shape=jax.ShapeDtypeStruct((m, dout), jnp.float32),
    )(h, p0, p1, w, b.reshape(1, dout))


def _mlp(h, w1, b1, w2, b2, bm):
    m, k = h.shape
    dh = w1.shape[1]
    dout = w2.shape[1]
    return pl.pallas_call(
        _mlp_body,
        grid=(m // bm,),
        in_specs=[
            pl.BlockSpec((bm, k), lambda i: (i, 0)),
            pl.BlockSpec((k, dh), lambda i: (0, 0)),
            pl.BlockSpec((1, dh), lambda i: (0, 0)),
            pl.BlockSpec((dh, dout), lambda i: (0, 0)),
            pl.BlockSpec((1, dout), lambda i: (0, 0)),
        ],
        out_specs=pl.BlockSpec((bm, dout), lambda i: (i, 0)),
        out_shape=jax.ShapeDtypeStruct((m, dout), jnp.float32),
    )(h, w1, b1.reshape(1, dh), w2, b2.reshape(1, dout))


_NC = 2
_NS = 16
_LANES = 16


def _make_message_pass(n, dh, e, chunk):
    assert e % _NS == 0
    e_per_t = e // _NS
    assert e_per_t % chunk == 0 and chunk % 8 == 0 and chunk <= 128
    n_chunks = e_per_t // chunk
    assert n_chunks % 4 == 0
    np_rows = ((n + 8 * _NS - 1) // (8 * _NS)) * (8 * _NS)
    rows_per_tile = np_rows // _NS
    nvec = dh // _LANES
    unroll = 8
    assert chunk % unroll == 0

    mesh = plsc.VectorSubcoreMesh(core_axis_name="c", subcore_axis_name="s")

    @functools.partial(
        pl.kernel,
        mesh=mesh,
        compiler_params=pltpu.CompilerParams(
            use_tc_tiling_on_sc=False, needs_layout_passes=False),
        out_type=jax.ShapeDtypeStruct((_NC, np_rows, dh), jnp.float32),
        scratch_types=(
            [
                pltpu.VMEM_SHARED((np_rows, dh), jnp.float32),
                pltpu.VMEM((n_chunks, chunk), jnp.int32),
            ]
            + [pltpu.VMEM((chunk,), jnp.int32)] * 4
            + [pltpu.VMEM((chunk, dh // 2), jnp.uint32)] * 4
            + [pltpu.VMEM((chunk, dh // 2), jnp.uint32)] * 4
            + [pltpu.VMEM((chunk, dh), jnp.float32)] * 4
            + [pltpu.SemaphoreType.DMA] * 16
        ),
    )
    def msg_kernel(ht0_hbm, ht1_hbm, et0_hbm, et1_hbm, src_hbm, dst_hbm,
                   z_hbm, out_hbm, aggr, src_v, *rest):
        dbuf = rest[0:4]
        erows = rest[4:8]
        hrows = rest[8:12]
        mbuf = rest[12:16]
        sem_d = rest[16:20]
        sem_e = rest[20:24]
        sem_h = rest[24:28]
        sem_s = rest[28:32]
        core = lax.axis_index("c")
        sub = lax.axis_index("s")

        pltpu.sync_copy(
            z_hbm.at[pl.ds(sub * rows_per_tile, rows_per_tile)],
            aggr.at[pl.ds(sub * rows_per_tile, rows_per_tile)],
        )
        plsc.subcore_barrier()

        pltpu.sync_copy(src_hbm.at[pl.ds(sub * n_chunks, n_chunks)], src_v)

        def run_edges(ht_hbm, et_hbm):
            def issue_loads(g, b):
                base = sub * e_per_t + g * chunk
                pltpu.async_copy(dst_hbm.at[pl.ds(base, chunk)], dbuf[b],
                                 sem_d[b])
                pltpu.async_copy(et_hbm.at[pl.ds(base, chunk)], erows[b],
                                 sem_e[b])
                pltpu.async_copy(ht_hbm.at[src_v.at[g]], hrows[b], sem_h[b])

            def wait_loads(g, b):
                base = sub * e_per_t + g * chunk
                pltpu.make_async_copy(dst_hbm.at[pl.ds(base, chunk)], dbuf[b],
                                      sem_d[b]).wait()
                pltpu.make_async_copy(et_hbm.at[pl.ds(base, chunk)], erows[b],
                                      sem_e[b]).wait()
                pltpu.make_async_copy(ht_hbm.at[src_v.at[g]], hrows[b],
                                      sem_h[b]).wait()

            def wait_scatter(b):
                pltpu.make_async_copy(mbuf[b], aggr.at[dbuf[b]],
                                      sem_s[b]).wait()

            def step(g, b):
                pb = (b + 2) % 4
                wait_scatter(pb)

                @pl.when(g + 2 < n_chunks)
                def _():
                    issue_loads(g + 2, pb)

                wait_loads(g, b)

                eb = erows[b]
                hb = hrows[b]
                mb = mbuf[b]

                def edge_body(i, _):
                    for u in range(unroll):
                        ii = i * unroll + u
                        for j in range(dh // 32):
                            sw = pl.ds(j * _LANES, _LANES)
                            wh = hb[ii, sw]
                            we = eb[ii, sw]
                            h_lo = plsc.bitcast(wh << 16, jnp.float32)
                            h_hi = plsc.bitcast(
                                wh & jnp.uint32(0xFFFF0000), jnp.float32)
                            e_lo = plsc.bitcast(we << 16, jnp.float32)
                            e_hi = plsc.bitcast(
                                we & jnp.uint32(0xFFFF0000), jnp.float32)
                            mb[ii, pl.ds(j * 32, _LANES)] = jnp.maximum(
                                h_lo + e_lo, 0.0)
                            mb[ii, pl.ds(j * 32 + _LANES, _LANES)] = (
                                jnp.maximum(h_hi + e_hi, 0.0))
                    return 0
                lax.fori_loop(0, chunk // unroll, edge_body, 0)

                pltpu.async_copy(mb, aggr.at[dbuf[b]], sem_s[b], add=True)

            pltpu.async_copy(z_hbm.at[pl.ds(0, chunk)], mbuf[2], sem_s[2])
            pltpu.async_copy(z_hbm.at[pl.ds(0, chunk)], mbuf[3], sem_s[3])
            issue_loads(0, 0)
            issue_loads(1, 1)

            def pipe_body(gp, _):
                for b in range(4):
                    step(gp * 4 + b, b)
                return 0

            lax.fori_loop(0, n_chunks // 4, pipe_body, 0)
            for t in range(n_chunks % 4):
                step(n_chunks - (n_chunks % 4) + t, t)
            for t in range(1, 3):
                wait_scatter((n_chunks - t) % 4)

        @pl.when(core == 0)
        def _():
            run_edges(ht0_hbm, et0_hbm)

        @pl.when(core == 1)
        def _():
            run_edges(ht1_hbm, et1_hbm)

        plsc.subcore_barrier()
        pltpu.sync_copy(
            aggr.at[pl.ds(sub * rows_per_tile, rows_per_tile)],
            out_hbm.at[core, pl.ds(sub * rows_per_tile, rows_per_tile)],
        )

    return msg_kernel, np_rows


def kernel(x, edge_index, edge_attr, W_in, b_in, W_nbr, W_e, W_self, b_conv,
           W_m1, b_m1, W_m2, b_m2):
    n, d = x.shape
    e = edge_index.shape[1]
    num_layers = W_nbr.shape[0]
    src = edge_index[0]
    dst = edge_index[1]

    bm = 1000
    chunk = 128

    gran = _NS * chunk * 4
    e_pad = ((e + gran - 1) // gran) * gran
    msg_pass, np_rows = _make_message_pass(n, d // 2, e_pad, chunk=chunk)
    if e_pad != e:
        pad = e_pad - e
        src = jnp.concatenate([src, jnp.zeros((pad,), jnp.int32)])
        dst = jnp.concatenate([dst, jnp.full((pad,), n, jnp.int32)])
        edge_attr = jnp.concatenate(
            [edge_attr, jnp.zeros((pad, edge_attr.shape[1]), jnp.float32)])
    be = 4096
    src = src.reshape(e_pad // chunk, chunk)

    zeros_np = jnp.zeros((np_rows, d // 2), jnp.float32)

    ets = [_mm_halves(edge_attr, W_e[l], be) for l in range(num_layers)]

    h = _mm_bias(x, W_in, b_in, bm)
    for l in range(num_layers):
        ht0, ht1 = _mm_halves(h, W_nbr[l], bm)
        et0, et1 = ets[l]
        partials = msg_pass(ht0, ht1, et0, et1, src, dst, zeros_np)
        p0 = partials[0, :n]
        p1 = partials[1, :n]
        h = _resid_mm(h, p0, p1, W_self[l], b_conv[l], bm)

    out = _mlp(h, W_m1, b_m1, W_m2, b_m2, bm)
    return out[:, 0]


# speedup vs baseline: 1.2290x; 1.0727x over previous
import functools

import jax
import jax.numpy as jnp
import numpy as np
from jax import lax
from jax.experimental import pallas as pl
from jax.experimental.pallas import tpu as pltpu
from jax.experimental.pallas import tpu_sc as plsc


def _mm_bias_body(a_ref, w_ref, b_ref, o_ref):
    o_ref[...] = (
        jnp.dot(a_ref[...], w_ref[...], preferred_element_type=jnp.float32)
        + b_ref[...]
    )


def _mm_halves_body(a_ref, w_ref, o0_ref, o1_ref):
    y = jnp.dot(a_ref[...], w_ref[...], preferred_element_type=jnp.float32)
    half = y.shape[1] // 2

    def pack_half(off):
        words = []
        for g in range(half // 32):
            blk_a = y[:, off + 32 * g: off + 32 * g + 16]
            blk_b = y[:, off + 32 * g + 16: off + 32 * g + 32]
            a_bits = lax.bitcast_convert_type(
                blk_a.astype(jnp.bfloat16), jnp.uint16).astype(jnp.uint32)
            b_bits = lax.bitcast_convert_type(
                blk_b.astype(jnp.bfloat16), jnp.uint16).astype(jnp.uint32)
            words.append(a_bits | (b_bits << 16))
        return jnp.concatenate(words, axis=1)

    o0_ref[...] = pack_half(0)
    o1_ref[...] = pack_half(half)


def _resid_mm_body(h_ref, p0_ref, p1_ref, w_ref, b_ref, o_ref):
    half = w_ref.shape[0] // 2
    aggr_mm = (
        jnp.dot(p0_ref[...], w_ref[:half, :], preferred_element_type=jnp.float32)
        + jnp.dot(p1_ref[...], w_ref[half:, :], preferred_element_type=jnp.float32)
    )
    o_ref[...] = h_ref[...] + aggr_mm + b_ref[...]


def _mlp_body(h_ref, w1_ref, b1_ref, w2_ref, b2_ref, o_ref):
    hid = jnp.maximum(
        jnp.dot(h_ref[...], w1_ref[...], preferred_element_type=jnp.float32)
        + b1_ref[...],
        0.0,
    )
    o_ref[...] = (
        jnp.dot(hid, w2_ref[...], preferred_element_type=jnp.float32) + b2_ref[...]
    )


def _mm_bias(a, w, b, bm):
    m, k = a.shape
    dout = w.shape[1]
    return pl.pallas_call(
        _mm_bias_body,
        grid=(m // bm,),
        in_specs=[
            pl.BlockSpec((bm, k), lambda i: (i, 0)),
            pl.BlockSpec((k, dout), lambda i: (0, 0)),
            pl.BlockSpec((1, dout), lambda i: (0, 0)),
        ],
        out_specs=pl.BlockSpec((bm, dout), lambda i: (i, 0)),
        out_shape=jax.ShapeDtypeStruct((m, dout), jnp.float32),
    )(a, w, b.reshape(1, dout))


def _mm_halves(a, w, bm):
    m, k = a.shape
    dout = w.shape[1]
    quarter = dout // 4
    return pl.pallas_call(
        _mm_halves_body,
        grid=(m // bm,),
        in_specs=[
            pl.BlockSpec((bm, k), lambda i: (i, 0)),
            pl.BlockSpec((k, dout), lambda i: (0, 0)),
        ],
        out_specs=(
            pl.BlockSpec((bm, quarter), lambda i: (i, 0)),
            pl.BlockSpec((bm, quarter), lambda i: (i, 0)),
        ),
        out_shape=(
            jax.ShapeDtypeStruct((m, quarter), jnp.uint32),
            jax.ShapeDtypeStruct((m, quarter), jnp.uint32),
        ),
    )(a, w)


def _resid_mm(h, p0, p1, w, b, bm):
    m, k = h.shape
    dout = w.shape[1]
    half = k // 2
    return pl.pallas_call(
        _resid_mm_body,
        grid=(m // bm,),
        in_specs=[
            pl.BlockSpec((bm, k), lambda i: (i, 0)),
            pl.BlockSpec((bm, half), lambda i: (i, 0)),
            pl.BlockSpec((bm, half), lambda i: (i, 0)),
            pl.BlockSpec((k, dout), lambda i: (0, 0)),
            pl.BlockSpec((1, dout), lambda i: (0, 0)),
        ],
        out_specs=pl.BlockSpec((bm, dout), lambda i: (i, 0)),
        out_shape=jax.ShapeDtypeStruct((m, dout), jnp.float32),
    )(h, p0, p1, w, b.reshape(1, dout))


def _mlp(h, w1, b1, w2, b2, bm):
    m, k = h.shape
    dh = w1.shape[1]
    dout = w2.shape[1]
    return pl.pallas_call(
        _mlp_body,
        grid=(m // bm,),
        in_specs=[
            pl.BlockSpec((bm, k), lambda i: (i, 0)),
            pl.BlockSpec((k, dh), lambda i: (0, 0)),
            pl.BlockSpec((1, dh), lambda i: (0, 0)),
            pl.BlockSpec((dh, dout), lambda i: (0, 0)),
            pl.BlockSpec((1, dout), lambda i: (0, 0)),
        ],
        out_specs=pl.BlockSpec((bm, dout), lambda i: (i, 0)),
        out_shape=jax.ShapeDtypeStruct((m, dout), jnp.float32),
    )(h, w1, b1.reshape(1, dh), w2, b2.reshape(1, dout))


_NC = 2
_NS = 16
_LANES = 16


def _make_message_pass(n, dh, e, chunk):
    assert e % _NS == 0
    e_per_t = e // _NS
    assert e_per_t % chunk == 0 and chunk % 8 == 0 and chunk <= 128
    n_chunks = e_per_t // chunk
    assert n_chunks % 4 == 0
    np_rows = ((n + 8 * _NS - 1) // (8 * _NS)) * (8 * _NS)
    rows_per_tile = np_rows // _NS
    nvec = dh // _LANES
    unroll = 8
    assert chunk % unroll == 0

    mesh = plsc.VectorSubcoreMesh(core_axis_name="c", subcore_axis_name="s")

    @functools.partial(
        pl.kernel,
        mesh=mesh,
        compiler_params=pltpu.CompilerParams(
            use_tc_tiling_on_sc=False, needs_layout_passes=False),
        out_type=jax.ShapeDtypeStruct((_NC, np_rows, dh), jnp.float32),
        scratch_types=(
            [
                pltpu.VMEM_SHARED((np_rows, dh), jnp.float32),
                pltpu.VMEM((n_chunks, chunk), jnp.int32),
            ]
            + [pltpu.VMEM((chunk,), jnp.int32)] * 4
            + [pltpu.VMEM((chunk, dh // 2), jnp.uint32)] * 4
            + [pltpu.VMEM((chunk, dh // 2), jnp.uint32)] * 4
            + [pltpu.VMEM((chunk, dh), jnp.float32)] * 4
            + [pltpu.SemaphoreType.DMA] * 16
        ),
    )
    def msg_kernel(ht0_hbm, ht1_hbm, et0_hbm, et1_hbm, src_hbm, dst_hbm,
                   z_hbm, out_hbm, aggr, src_v, *rest):
        dbuf = rest[0:4]
        erows = rest[4:8]
        hrows = rest[8:12]
        mbuf = rest[12:16]
        sem_d = rest[16:20]
        sem_e = rest[20:24]
        sem_h = rest[24:28]
        sem_s = rest[28:32]
        core = lax.axis_index("c")
        sub = lax.axis_index("s")

        pltpu.sync_copy(
            z_hbm.at[pl.ds(sub * rows_per_tile, rows_per_tile)],
            aggr.at[pl.ds(sub * rows_per_tile, rows_per_tile)],
        )
        plsc.subcore_barrier()

        pltpu.sync_copy(src_hbm.at[pl.ds(sub * n_chunks, n_chunks)], src_v)

        def run_edges(ht_hbm, et_hbm):
            def issue_loads(g, b):
                base = sub * e_per_t + g * chunk
                pltpu.async_copy(dst_hbm.at[pl.ds(base, chunk)], dbuf[b],
                                 sem_d[b])
                pltpu.async_copy(et_hbm.at[pl.ds(base, chunk)], erows[b],
                                 sem_e[b])
                pltpu.async_copy(ht_hbm.at[src_v.at[g]], hrows[b], sem_h[b])

            def wait_loads(g, b):
                base = sub * e_per_t + g * chunk
                pltpu.make_async_copy(dst_hbm.at[pl.ds(base, chunk)], dbuf[b],
                                      sem_d[b]).wait()
                pltpu.make_async_copy(et_hbm.at[pl.ds(base, chunk)], erows[b],
                                      sem_e[b]).wait()
                pltpu.make_async_copy(ht_hbm.at[src_v.at[g]], hrows[b],
                                      sem_h[b]).wait()

            def wait_scatter(b):
                pltpu.make_async_copy(mbuf[b], aggr.at[dbuf[b]],
                                      sem_s[b]).wait()

            def step(g, b):
                pb = (b + 2) % 4
                wait_scatter(pb)

                @pl.when(g + 2 < n_chunks)
                def _():
                    issue_loads(g + 2, pb)

                wait_loads(g, b)

                eb = erows[b]
                hb = hrows[b]
                mb = mbuf[b]

                def edge_body(i, _):
                    for u in range(unroll):
                        ii = i * unroll + u
                        for j in range(dh // 32):
                            sw = pl.ds(j * _LANES, _LANES)
                            wh = hb[ii, sw]
                            we = eb[ii, sw]
                            h_lo = plsc.bitcast(wh << 16, jnp.float32)
                            h_hi = plsc.bitcast(
                                wh & jnp.uint32(0xFFFF0000), jnp.float32)
                            e_lo = plsc.bitcast(we << 16, jnp.float32)
                            e_hi = plsc.bitcast(
                                we & jnp.uint32(0xFFFF0000), jnp.float32)
                            mb[ii, pl.ds(j * 32, _LANES)] = jnp.maximum(
                                h_lo + e_lo, 0.0)
                            mb[ii, pl.ds(j * 32 + _LANES, _LANES)] = (
                                jnp.maximum(h_hi + e_hi, 0.0))
                    return 0
                lax.fori_loop(0, chunk // unroll, edge_body, 0)

                pltpu.async_copy(mb, aggr.at[dbuf[b]], sem_s[b], add=True)

            pltpu.async_copy(z_hbm.at[pl.ds(0, chunk)], mbuf[2], sem_s[2])
            pltpu.async_copy(z_hbm.at[pl.ds(0, chunk)], mbuf[3], sem_s[3])
            issue_loads(0, 0)
            issue_loads(1, 1)

            def pipe_body(gp, _):
                for b in range(4):
                    step(gp * 4 + b, b)
                return 0

            lax.fori_loop(0, n_chunks // 4, pipe_body, 0)
            for t in range(n_chunks % 4):
                step(n_chunks - (n_chunks % 4) + t, t)
            for t in range(1, 3):
                wait_scatter((n_chunks - t) % 4)

        @pl.when(core == 0)
        def _():
            run_edges(ht0_hbm, et0_hbm)

        @pl.when(core == 1)
        def _():
            run_edges(ht1_hbm, et1_hbm)

        plsc.subcore_barrier()
        pltpu.sync_copy(
            aggr.at[pl.ds(sub * rows_per_tile, rows_per_tile)],
            out_hbm.at[core, pl.ds(sub * rows_per_tile, rows_per_tile)],
        )

    return msg_kernel, np_rows


def kernel(x, edge_index, edge_attr, W_in, b_in, W_nbr, W_e, W_self, b_conv,
           W_m1, b_m1, W_m2, b_m2):
    n, d = x.shape
    e = edge_index.shape[1]
    num_layers = W_nbr.shape[0]
    src = edge_index[0]
    dst = edge_index[1]

    bm = 1000
    chunk = 112

    gran = _NS * chunk * 4
    e_pad = ((e + gran - 1) // gran) * gran
    msg_pass, np_rows = _make_message_pass(n, d // 2, e_pad, chunk=chunk)
    if e_pad != e:
        pad = e_pad - e
        src = jnp.concatenate([src, jnp.zeros((pad,), jnp.int32)])
        dst = jnp.concatenate([dst, jnp.full((pad,), n, jnp.int32)])
        edge_attr = jnp.concatenate(
            [edge_attr, jnp.zeros((pad, edge_attr.shape[1]), jnp.float32)])
    assert e_pad % 64 == 0
    be = e_pad // 64
    src = src.reshape(e_pad // chunk, chunk)

    zeros_np = jnp.zeros((np_rows, d // 2), jnp.float32)

    ets = [_mm_halves(edge_attr, W_e[l], be) for l in range(num_layers)]

    h = _mm_bias(x, W_in, b_in, bm)
    for l in range(num_layers):
        ht0, ht1 = _mm_halves(h, W_nbr[l], bm)
        et0, et1 = ets[l]
        partials = msg_pass(ht0, ht1, et0, et1, src, dst, zeros_np)
        p0 = partials[0, :n]
        p1 = partials[1, :n]
        h = _resid_mm(h, p0, p1, W_self[l], b_conv[l], bm)

    out = _mlp(h, W_m1, b_m1, W_m2, b_m2, bm)
    return out[:, 0]


# speedup vs baseline: 1.2298x; 1.0007x over previous
import functools

import jax
import jax.numpy as jnp
import numpy as np
from jax import lax
from jax.experimental import pallas as pl
from jax.experimental.pallas import tpu as pltpu
from jax.experimental.pallas import tpu_sc as plsc


def _mm_bias_body(a_ref, w_ref, b_ref, o_ref):
    o_ref[...] = (
        jnp.dot(a_ref[...], w_ref[...], preferred_element_type=jnp.float32)
        + b_ref[...]
    )


def _mm_halves_body(a_ref, w_ref, o0_ref, o1_ref):
    y = jnp.dot(a_ref[...], w_ref[...], preferred_element_type=jnp.float32)
    half = y.shape[1] // 2

    def pack_half(off):
        words = []
        for g in range(half // 32):
            blk_a = y[:, off + 32 * g: off + 32 * g + 16]
            blk_b = y[:, off + 32 * g + 16: off + 32 * g + 32]
            a_bits = lax.bitcast_convert_type(
                blk_a.astype(jnp.bfloat16), jnp.uint16).astype(jnp.uint32)
            b_bits = lax.bitcast_convert_type(
                blk_b.astype(jnp.bfloat16), jnp.uint16).astype(jnp.uint32)
            words.append(a_bits | (b_bits << 16))
        return jnp.concatenate(words, axis=1)

    o0_ref[...] = pack_half(0)
    o1_ref[...] = pack_half(half)


def _resid_mm_body(h_ref, p0_ref, p1_ref, w_ref, b_ref, o_ref):
    half = w_ref.shape[0] // 2
    aggr_mm = (
        jnp.dot(p0_ref[...], w_ref[:half, :], preferred_element_type=jnp.float32)
        + jnp.dot(p1_ref[...], w_ref[half:, :], preferred_element_type=jnp.float32)
    )
    o_ref[...] = h_ref[...] + aggr_mm + b_ref[...]


def _mlp_body(h_ref, w1_ref, b1_ref, w2_ref, b2_ref, o_ref):
    hid = jnp.maximum(
        jnp.dot(h_ref[...], w1_ref[...], preferred_element_type=jnp.float32)
        + b1_ref[...],
        0.0,
    )
    o_ref[...] = (
        jnp.dot(hid, w2_ref[...], preferred_element_type=jnp.float32) + b2_ref[...]
    )


def _mm_bias(a, w, b, bm):
    m, k = a.shape
    dout = w.shape[1]
    return pl.pallas_call(
        _mm_bias_body,
        grid=(m // bm,),
        in_specs=[
            pl.BlockSpec((bm, k), lambda i: (i, 0)),
            pl.BlockSpec((k, dout), lambda i: (0, 0)),
            pl.BlockSpec((1, dout), lambda i: (0, 0)),
        ],
        out_specs=pl.BlockSpec((bm, dout), lambda i: (i, 0)),
        out_shape=jax.ShapeDtypeStruct((m, dout), jnp.float32),
    )(a, w, b.reshape(1, dout))


def _mm_halves(a, w, bm):
    m, k = a.shape
    dout = w.shape[1]
    quarter = dout // 4
    return pl.pallas_call(
        _mm_halves_body,
        grid=(m // bm,),
        in_specs=[
            pl.BlockSpec((bm, k), lambda i: (i, 0)),
            pl.BlockSpec((k, dout), lambda i: (0, 0)),
        ],
        out_specs=(
            pl.BlockSpec((bm, quarter), lambda i: (i, 0)),
            pl.BlockSpec((bm, quarter), lambda i: (i, 0)),
        ),
        out_shape=(
            jax.ShapeDtypeStruct((m, quarter), jnp.uint32),
            jax.ShapeDtypeStruct((m, quarter), jnp.uint32),
        ),
    )(a, w)


def _resid_mm(h, p0, p1, w, b, bm):
    m, k = h.shape
    dout = w.shape[1]
    half = k // 2
    return pl.pallas_call(
        _resid_mm_body,
        grid=(m // bm,),
        in_specs=[
            pl.BlockSpec((bm, k), lambda i: (i, 0)),
            pl.BlockSpec((bm, half), lambda i: (i, 0)),
            pl.BlockSpec((bm, half), lambda i: (i, 0)),
            pl.BlockSpec((k, dout), lambda i: (0, 0)),
            pl.BlockSpec((1, dout), lambda i: (0, 0)),
        ],
        out_specs=pl.BlockSpec((bm, dout), lambda i: (i, 0)),
        out_shape=jax.ShapeDtypeStruct((m, dout), jnp.float32),
    )(h, p0, p1, w, b.reshape(1, dout))


def _mlp(h, w1, b1, w2, b2, bm):
    m, k = h.shape
    dh = w1.shape[1]
    dout = w2.shape[1]
    return pl.pallas_call(
        _mlp_body,
        grid=(m // bm,),
        in_specs=[
            pl.BlockSpec((bm, k), lambda i: (i, 0)),
            pl.BlockSpec((k, dh), lambda i: (0, 0)),
            pl.BlockSpec((1, dh), lambda i: (0, 0)),
            pl.BlockSpec((dh, dout), lambda i: (0, 0)),
            pl.BlockSpec((1, dout), lambda i: (0, 0)),
        ],
        out_specs=pl.BlockSpec((bm, dout), lambda i: (i, 0)),
        out_shape=jax.ShapeDtypeStruct((m, dout), jnp.float32),
    )(h, w1, b1.reshape(1, dh), w2, b2.reshape(1, dout))


_NC = 2
_NS = 16
_LANES = 16


def _make_message_pass(n, dh, e, chunk):
    assert e % _NS == 0
    e_per_t = e // _NS
    assert e_per_t % chunk == 0 and chunk % 8 == 0 and chunk <= 128
    n_chunks = e_per_t // chunk
    assert n_chunks % 4 == 0
    np_rows = ((n + 8 * _NS - 1) // (8 * _NS)) * (8 * _NS)
    rows_per_tile = np_rows // _NS
    nvec = dh // _LANES
    unroll = 8
    assert chunk % unroll == 0

    mesh = plsc.VectorSubcoreMesh(core_axis_name="c", subcore_axis_name="s")

    @functools.partial(
        pl.kernel,
        mesh=mesh,
        compiler_params=pltpu.CompilerParams(
            use_tc_tiling_on_sc=False, needs_layout_passes=False),
        out_type=jax.ShapeDtypeStruct((_NC, np_rows, dh), jnp.float32),
        scratch_types=(
            [
                pltpu.VMEM_SHARED((np_rows, dh), jnp.float32),
                pltpu.VMEM((n_chunks, chunk), jnp.int32),
            ]
            + [pltpu.VMEM((chunk,), jnp.int32)] * 4
            + [pltpu.VMEM((chunk, dh // 2), jnp.uint32)] * 4
            + [pltpu.VMEM((chunk, dh // 2), jnp.uint32)] * 4
            + [pltpu.VMEM((chunk, dh), jnp.float32)] * 4
            + [pltpu.SemaphoreType.DMA] * 16
        ),
    )
    def msg_kernel(ht0_hbm, ht1_hbm, et0_hbm, et1_hbm, src_hbm, dst_hbm,
                   z_hbm, out_hbm, aggr, src_v, *rest):
        dbuf = rest[0:4]
        erows = rest[4:8]
        hrows = rest[8:12]
        mbuf = rest[12:16]
        sem_d = rest[16:20]
        sem_e = rest[20:24]
        sem_h = rest[24:28]
        sem_s = rest[28:32]
        core = lax.axis_index("c")
        sub = lax.axis_index("s")

        pltpu.sync_copy(
            z_hbm.at[pl.ds(sub * rows_per_tile, rows_per_tile)],
            aggr.at[pl.ds(sub * rows_per_tile, rows_per_tile)],
        )
        plsc.subcore_barrier()

        pltpu.sync_copy(src_hbm.at[pl.ds(sub * n_chunks, n_chunks)], src_v)

        def run_edges(ht_hbm, et_hbm):
            def issue_loads(g, b):
                base = sub * e_per_t + g * chunk
                pltpu.async_copy(dst_hbm.at[pl.ds(base, chunk)], dbuf[b],
                                 sem_d[b])
                pltpu.async_copy(et_hbm.at[pl.ds(base, chunk)], erows[b],
                                 sem_e[b])
                pltpu.async_copy(ht_hbm.at[src_v.at[g]], hrows[b], sem_h[b])

            def wait_loads(g, b):
                base = sub * e_per_t + g * chunk
                pltpu.make_async_copy(dst_hbm.at[pl.ds(base, chunk)], dbuf[b],
                                      sem_d[b]).wait()
                pltpu.make_async_copy(et_hbm.at[pl.ds(base, chunk)], erows[b],
                                      sem_e[b]).wait()
                pltpu.make_async_copy(ht_hbm.at[src_v.at[g]], hrows[b],
                                      sem_h[b]).wait()

            def wait_scatter(b):
                pltpu.make_async_copy(mbuf[b], aggr.at[dbuf[b]],
                                      sem_s[b]).wait()

            def step(g, b):
                pb = (b + 2) % 4
                wait_scatter(pb)

                @pl.when(g + 2 < n_chunks)
                def _():
                    issue_loads(g + 2, pb)

                wait_loads(g, b)

                eb = erows[b]
                hb = hrows[b]
                mb = mbuf[b]

                def edge_body(i, _):
                    for u in range(unroll):
                        ii = i * unroll + u
                        for j in range(dh // 32):
                            sw = pl.ds(j * _LANES, _LANES)
                            wh = hb[ii, sw]
                            we = eb[ii, sw]
                            h_lo = plsc.bitcast(wh << 16, jnp.float32)
                            h_hi = plsc.bitcast(
                                wh & jnp.uint32(0xFFFF0000), jnp.float32)
                            e_lo = plsc.bitcast(we << 16, jnp.float32)
                            e_hi = plsc.bitcast(
                                we & jnp.uint32(0xFFFF0000), jnp.float32)
                            mb[ii, pl.ds(j * 32, _LANES)] = jnp.maximum(
                                h_lo + e_lo, 0.0)
                            mb[ii, pl.ds(j * 32 + _LANES, _LANES)] = (
                                jnp.maximum(h_hi + e_hi, 0.0))
                    return 0
                lax.fori_loop(0, chunk // unroll, edge_body, 0)

                pltpu.async_copy(mb, aggr.at[dbuf[b]], sem_s[b], add=True)

            pltpu.async_copy(z_hbm.at[pl.ds(0, chunk)], mbuf[2], sem_s[2])
            pltpu.async_copy(z_hbm.at[pl.ds(0, chunk)], mbuf[3], sem_s[3])
            issue_loads(0, 0)
            issue_loads(1, 1)

            def pipe_body(gp, _):
                for b in range(4):
                    step(gp * 4 + b, b)
                return 0

            lax.fori_loop(0, n_chunks // 4, pipe_body, 0)
            for t in range(n_chunks % 4):
                step(n_chunks - (n_chunks % 4) + t, t)
            for t in range(1, 3):
                wait_scatter((n_chunks - t) % 4)

        @pl.when(core == 0)
        def _():
            run_edges(ht0_hbm, et0_hbm)

        @pl.when(core == 1)
        def _():
            run_edges(ht1_hbm, et1_hbm)

        plsc.subcore_barrier()
        pltpu.sync_copy(
            aggr.at[pl.ds(sub * rows_per_tile, rows_per_tile)],
            out_hbm.at[core, pl.ds(sub * rows_per_tile, rows_per_tile)],
        )

    return msg_kernel, np_rows


def kernel(x, edge_index, edge_attr, W_in, b_in, W_nbr, W_e, W_self, b_conv,
           W_m1, b_m1, W_m2, b_m2):
    n, d = x.shape
    e = edge_index.shape[1]
    num_layers = W_nbr.shape[0]
    src = edge_index[0]
    dst = edge_index[1]

    bm = 1000
    chunk = 80

    gran = _NS * chunk * 4
    e_pad = ((e + gran - 1) // gran) * gran
    msg_pass, np_rows = _make_message_pass(n, d // 2, e_pad, chunk=chunk)
    if e_pad != e:
        pad = e_pad - e
        src = jnp.concatenate([src, jnp.zeros((pad,), jnp.int32)])
        dst = jnp.concatenate([dst, jnp.full((pad,), n, jnp.int32)])
        edge_attr = jnp.concatenate(
            [edge_attr, jnp.zeros((pad, edge_attr.shape[1]), jnp.float32)])
    assert e_pad % 64 == 0
    be = e_pad // 64
    src = src.reshape(e_pad // chunk, chunk)

    zeros_np = jnp.zeros((np_rows, d // 2), jnp.float32)

    ets = [_mm_halves(edge_attr, W_e[l], be) for l in range(num_layers)]

    h = _mm_bias(x, W_in, b_in, bm)
    for l in range(num_layers):
        ht0, ht1 = _mm_halves(h, W_nbr[l], bm)
        et0, et1 = ets[l]
        partials = msg_pass(ht0, ht1, et0, et1, src, dst, zeros_np)
        p0 = partials[0, :n]
        p1 = partials[1, :n]
        h = _resid_mm(h, p0, p1, W_self[l], b_conv[l], bm)

    out = _mlp(h, W_m1, b_m1, W_m2, b_m2, bm)
    return out[:, 0]


# speedup vs baseline: 1.2921x; 1.0506x over previous
import functools

import jax
import jax.numpy as jnp
import numpy as np
from jax import lax
from jax.experimental import pallas as pl
from jax.experimental.pallas import tpu as pltpu
from jax.experimental.pallas import tpu_sc as plsc


def _mm_bias_body(a_ref, w_ref, b_ref, o_ref):
    o_ref[...] = (
        jnp.dot(a_ref[...], w_ref[...], preferred_element_type=jnp.float32)
        + b_ref[...]
    )


def _mm_halves_body(a_ref, w_ref, o0_ref, o1_ref):
    y = jnp.dot(a_ref[...], w_ref[...], preferred_element_type=jnp.float32)
    half = y.shape[1] // 2

    def pack_half(off):
        words = []
        for g in range(half // 32):
            blk_a = y[:, off + 32 * g: off + 32 * g + 16]
            blk_b = y[:, off + 32 * g + 16: off + 32 * g + 32]
            a_bits = lax.bitcast_convert_type(
                blk_a.astype(jnp.bfloat16), jnp.uint16).astype(jnp.uint32)
            b_bits = lax.bitcast_convert_type(
                blk_b.astype(jnp.bfloat16), jnp.uint16).astype(jnp.uint32)
            words.append(a_bits | (b_bits << 16))
        return jnp.concatenate(words, axis=1)

    o0_ref[...] = pack_half(0)
    o1_ref[...] = pack_half(half)


def _resid_mm_body(h_ref, p0_ref, p1_ref, w_ref, b_ref, o_ref):
    half = w_ref.shape[0] // 2
    aggr_mm = (
        jnp.dot(p0_ref[...], w_ref[:half, :], preferred_element_type=jnp.float32)
        + jnp.dot(p1_ref[...], w_ref[half:, :], preferred_element_type=jnp.float32)
    )
    o_ref[...] = h_ref[...] + aggr_mm + b_ref[...]


def _mlp_body(h_ref, w1_ref, b1_ref, w2_ref, b2_ref, o_ref):
    hid = jnp.maximum(
        jnp.dot(h_ref[...], w1_ref[...], preferred_element_type=jnp.float32)
        + b1_ref[...],
        0.0,
    )
    o_ref[...] = (
        jnp.dot(hid, w2_ref[...], preferred_element_type=jnp.float32) + b2_ref[...]
    )


def _mm_bias(a, w, b, bm):
    m, k = a.shape
    dout = w.shape[1]
    return pl.pallas_call(
        _mm_bias_body,
        grid=(m // bm,),
        in_specs=[
            pl.BlockSpec((bm, k), lambda i: (i, 0)),
            pl.BlockSpec((k, dout), lambda i: (0, 0)),
            pl.BlockSpec((1, dout), lambda i: (0, 0)),
        ],
        out_specs=pl.BlockSpec((bm, dout), lambda i: (i, 0)),
        out_shape=jax.ShapeDtypeStruct((m, dout), jnp.float32),
    )(a, w, b.reshape(1, dout))


def _mm_halves(a, w, bm):
    m, k = a.shape
    dout = w.shape[1]
    quarter = dout // 4
    return pl.pallas_call(
        _mm_halves_body,
        grid=(m // bm,),
        in_specs=[
            pl.BlockSpec((bm, k), lambda i: (i, 0)),
            pl.BlockSpec((k, dout), lambda i: (0, 0)),
        ],
        out_specs=(
            pl.BlockSpec((bm, quarter), lambda i: (i, 0)),
            pl.BlockSpec((bm, quarter), lambda i: (i, 0)),
        ),
        out_shape=(
            jax.ShapeDtypeStruct((m, quarter), jnp.uint32),
            jax.ShapeDtypeStruct((m, quarter), jnp.uint32),
        ),
    )(a, w)


def _resid_mm(h, p0, p1, w, b, bm):
    m, k = h.shape
    dout = w.shape[1]
    half = k // 2
    return pl.pallas_call(
        _resid_mm_body,
        grid=(m // bm,),
        in_specs=[
            pl.BlockSpec((bm, k), lambda i: (i, 0)),
            pl.BlockSpec((bm, half), lambda i: (i, 0)),
            pl.BlockSpec((bm, half), lambda i: (i, 0)),
            pl.BlockSpec((k, dout), lambda i: (0, 0)),
            pl.BlockSpec((1, dout), lambda i: (0, 0)),
        ],
        out_specs=pl.BlockSpec((bm, dout), lambda i: (i, 0)),
        out_shape=jax.ShapeDtypeStruct((m, dout), jnp.float32),
    )(h, p0, p1, w, b.reshape(1, dout))


def _mlp(h, w1, b1, w2, b2, bm):
    m, k = h.shape
    dh = w1.shape[1]
    dout = w2.shape[1]
    return pl.pallas_call(
        _mlp_body,
        grid=(m // bm,),
        in_specs=[
            pl.BlockSpec((bm, k), lambda i: (i, 0)),
            pl.BlockSpec((k, dh), lambda i: (0, 0)),
            pl.BlockSpec((1, dh), lambda i: (0, 0)),
            pl.BlockSpec((dh, dout), lambda i: (0, 0)),
            pl.BlockSpec((1, dout), lambda i: (0, 0)),
        ],
        out_specs=pl.BlockSpec((bm, dout), lambda i: (i, 0)),
        out_shape=jax.ShapeDtypeStruct((m, dout), jnp.float32),
    )(h, w1, b1.reshape(1, dh), w2, b2.reshape(1, dout))


_NC = 2
_NS = 16
_LANES = 16


def _make_message_pass(n, dh, e, chunk):
    assert e % _NS == 0
    e_per_t = e // _NS
    assert e_per_t % chunk == 0 and chunk % 8 == 0 and chunk <= 128
    n_chunks = e_per_t // chunk
    assert n_chunks >= 4
    np_rows = ((n + 8 * _NS - 1) // (8 * _NS)) * (8 * _NS)
    rows_per_tile = np_rows // _NS
    nvec = dh // _LANES
    unroll = 8
    assert chunk % unroll == 0

    mesh = plsc.VectorSubcoreMesh(core_axis_name="c", subcore_axis_name="s")

    @functools.partial(
        pl.kernel,
        mesh=mesh,
        compiler_params=pltpu.CompilerParams(
            use_tc_tiling_on_sc=False, needs_layout_passes=False),
        out_type=jax.ShapeDtypeStruct((_NC, np_rows, dh), jnp.float32),
        scratch_types=(
            [
                pltpu.VMEM_SHARED((np_rows, dh), jnp.float32),
                pltpu.VMEM((n_chunks, chunk), jnp.int32),
            ]
            + [pltpu.VMEM((chunk,), jnp.int32)] * 4
            + [pltpu.VMEM((chunk, dh // 2), jnp.uint32)] * 4
            + [pltpu.VMEM((chunk, dh // 2), jnp.uint32)] * 4
            + [pltpu.VMEM((chunk, dh), jnp.float32)] * 4
            + [pltpu.SemaphoreType.DMA] * 16
        ),
    )
    def msg_kernel(ht0_hbm, ht1_hbm, et0_hbm, et1_hbm, src_hbm, dst_hbm,
                   z_hbm, out_hbm, aggr, src_v, *rest):
        dbuf = rest[0:4]
        erows = rest[4:8]
        hrows = rest[8:12]
        mbuf = rest[12:16]
        sem_d = rest[16:20]
        sem_e = rest[20:24]
        sem_h = rest[24:28]
        sem_s = rest[28:32]
        core = lax.axis_index("c")
        sub = lax.axis_index("s")

        pltpu.sync_copy(
            z_hbm.at[pl.ds(sub * rows_per_tile, rows_per_tile)],
            aggr.at[pl.ds(sub * rows_per_tile, rows_per_tile)],
        )
        plsc.subcore_barrier()

        pltpu.sync_copy(src_hbm.at[pl.ds(sub * n_chunks, n_chunks)], src_v)

        def run_edges(ht_hbm, et_hbm):
            def issue_loads(g, b):
                base = sub * e_per_t + g * chunk
                pltpu.async_copy(dst_hbm.at[pl.ds(base, chunk)], dbuf[b],
                                 sem_d[b])
                pltpu.async_copy(et_hbm.at[pl.ds(base, chunk)], erows[b],
                                 sem_e[b])
                pltpu.async_copy(ht_hbm.at[src_v.at[g]], hrows[b], sem_h[b])

            def wait_loads(g, b):
                base = sub * e_per_t + g * chunk
                pltpu.make_async_copy(dst_hbm.at[pl.ds(base, chunk)], dbuf[b],
                                      sem_d[b]).wait()
                pltpu.make_async_copy(et_hbm.at[pl.ds(base, chunk)], erows[b],
                                      sem_e[b]).wait()
                pltpu.make_async_copy(ht_hbm.at[src_v.at[g]], hrows[b],
                                      sem_h[b]).wait()

            def wait_scatter(b):
                pltpu.make_async_copy(mbuf[b], aggr.at[dbuf[b]],
                                      sem_s[b]).wait()

            def step(g, b):
                pb = (b + 2) % 4
                wait_scatter(pb)

                @pl.when(g + 2 < n_chunks)
                def _():
                    issue_loads(g + 2, pb)

                wait_loads(g, b)

                eb = erows[b]
                hb = hrows[b]
                mb = mbuf[b]

                def edge_body(i, _):
                    for u in range(unroll):
                        ii = i * unroll + u
                        for j in range(dh // 32):
                            sw = pl.ds(j * _LANES, _LANES)
                            wh = hb[ii, sw]
                            we = eb[ii, sw]
                            h_lo = plsc.bitcast(wh << 16, jnp.float32)
                            h_hi = plsc.bitcast(
                                wh & jnp.uint32(0xFFFF0000), jnp.float32)
                            e_lo = plsc.bitcast(we << 16, jnp.float32)
                            e_hi = plsc.bitcast(
                                we & jnp.uint32(0xFFFF0000), jnp.float32)
                            mb[ii, pl.ds(j * 32, _LANES)] = jnp.maximum(
                                h_lo + e_lo, 0.0)
                            mb[ii, pl.ds(j * 32 + _LANES, _LANES)] = (
                                jnp.maximum(h_hi + e_hi, 0.0))
                    return 0
                lax.fori_loop(0, chunk // unroll, edge_body, 0)

                pltpu.async_copy(mb, aggr.at[dbuf[b]], sem_s[b], add=True)

            pltpu.async_copy(z_hbm.at[pl.ds(0, chunk)], mbuf[2], sem_s[2])
            pltpu.async_copy(z_hbm.at[pl.ds(0, chunk)], mbuf[3], sem_s[3])
            issue_loads(0, 0)
            issue_loads(1, 1)

            def pipe_body(gp, _):
                for b in range(4):
                    step(gp * 4 + b, b)
                return 0

            lax.fori_loop(0, n_chunks // 4, pipe_body, 0)
            for t in range(n_chunks % 4):
                step(n_chunks - (n_chunks % 4) + t, t)
            for t in range(1, 3):
                wait_scatter((n_chunks - t) % 4)

        @pl.when(core == 0)
        def _():
            run_edges(ht0_hbm, et0_hbm)

        @pl.when(core == 1)
        def _():
            run_edges(ht1_hbm, et1_hbm)

        plsc.subcore_barrier()
        pltpu.sync_copy(
            aggr.at[pl.ds(sub * rows_per_tile, rows_per_tile)],
            out_hbm.at[core, pl.ds(sub * rows_per_tile, rows_per_tile)],
        )

    return msg_kernel, np_rows


def kernel(x, edge_index, edge_attr, W_in, b_in, W_nbr, W_e, W_self, b_conv,
           W_m1, b_m1, W_m2, b_m2):
    n, d = x.shape
    e = edge_index.shape[1]
    num_layers = W_nbr.shape[0]
    src = edge_index[0]
    dst = edge_index[1]

    bm = 1000
    chunk = 80

    gran = _NS * chunk
    e_pad = ((e + gran - 1) // gran) * gran
    msg_pass, np_rows = _make_message_pass(n, d // 2, e_pad, chunk=chunk)
    if e_pad != e:
        pad = e_pad - e
        src = jnp.concatenate([src, jnp.zeros((pad,), jnp.int32)])
        dst = jnp.concatenate([dst, jnp.full((pad,), n, jnp.int32)])
        edge_attr = jnp.concatenate(
            [edge_attr, jnp.zeros((pad, edge_attr.shape[1]), jnp.float32)])
    assert e_pad % 64 == 0
    be = e_pad // 64
    src = src.reshape(e_pad // chunk, chunk)

    zeros_np = jnp.zeros((np_rows, d // 2), jnp.float32)

    ets = [_mm_halves(edge_attr, W_e[l], be) for l in range(num_layers)]

    h = _mm_bias(x, W_in, b_in, bm)
    for l in range(num_layers):
        ht0, ht1 = _mm_halves(h, W_nbr[l], bm)
        et0, et1 = ets[l]
        partials = msg_pass(ht0, ht1, et0, et1, src, dst, zeros_np)
        p0 = partials[0, :n]
        p1 = partials[1, :n]
        h = _resid_mm(h, p0, p1, W_self[l], b_conv[l], bm)

    out = _mlp(h, W_m1, b_m1, W_m2, b_m2, bm)
    return out[:, 0]
